# baseline (speedup 1.0000x reference)
import functools
import jax
import jax.numpy as jnp
from jax import lax
from jax.experimental import pallas as pl
from jax.experimental.pallas import tpu as pltpu, tpu_sc as plsc

VOCAB = 1000000
D_MODEL = 64
NC, NS = 2, 16
NW = NC * NS
L = 16


def _make_gather(B: int, S: int, D: int):
    assert B % NW == 0
    b_per_w = B // NW
    n_idx = b_per_w * S
    assert b_per_w % L == 0
    n_groups = b_per_w // L
    mesh = plsc.VectorSubcoreMesh(
        core_axis_name="c", subcore_axis_name="s",
        num_cores=NC, num_subcores=NS)

    @functools.partial(
        pl.kernel, mesh=mesh,
        out_type=jax.ShapeDtypeStruct((S, D, B), jnp.float32),
        compiler_params=pltpu.CompilerParams(
            use_tc_tiling_on_sc=False, needs_layout_passes=False),
        scratch_types=[
            pltpu.VMEM((n_idx,), jnp.int32),
            [pltpu.VMEM((b_per_w,), jnp.int32)] * 2,
            [pltpu.VMEM((b_per_w, D), jnp.float32)] * 2,
            pltpu.VMEM((D, b_per_w), jnp.float32),
            [pltpu.SemaphoreType.DMA] * 2,
            pltpu.SemaphoreType.DMA,
        ],
    )
    def k(table_hbm, idx_hbm, out_hbm, x_all, idx_s, rows, trans,
          gsem, ssem):
        wid = lax.axis_index("s") * NC + lax.axis_index("c")
        base = wid * b_per_w

        pltpu.sync_copy(idx_hbm.at[pl.ds(base * S, n_idx)], x_all)

        def build_idx(s, slot):
            offs0 = lax.iota(jnp.int32, L) * S
            for g in range(n_groups):
                offs = offs0 + (g * L * S + s)
                idx_s[slot][pl.ds(g * L, L)] = plsc.load_gather(
                    x_all, [offs])

        def gather(slot):
            return pltpu.async_copy(
                table_hbm.at[idx_s[slot]], rows[slot], gsem[slot])

        def gather_wait(slot):
            pltpu.make_async_copy(
                table_hbm.at[idx_s[slot]], rows[slot], gsem[slot]).wait()

        def transpose(slot):
            @pl.loop(0, D, unroll=4)
            def _(d):
                b_iota = lax.iota(jnp.int32, L)
                d_splat = jnp.full((L,), d, jnp.int32)
                for g in range(n_groups):
                    trans[d, pl.ds(g * L, L)] = plsc.load_gather(
                        rows[slot], [b_iota + g * L, d_splat])

        def store(s):
            return pltpu.async_copy(
                trans, out_hbm.at[s, :, pl.ds(base, b_per_w)], ssem)

        def store_wait(s):
            pltpu.make_async_copy(
                trans, out_hbm.at[s, :, pl.ds(base, b_per_w)], ssem).wait()

        build_idx(0, 0)
        gather(0)

        @pl.loop(0, S, step=2)
        def _(s0):
            for b in range(2):
                s = s0 + b
                q = 1 - b
                gather_wait(b)

                @pl.when(s + 1 < S)
                def _():
                    build_idx(s + 1, q)
                    gather(q)

                @pl.when(s > 0)
                def _():
                    store_wait(s - 1)
                transpose(b)
                store(s)

        store_wait(S - 1)

    return k


def kernel(x, table):
    B, S = x.shape
    D = table.shape[1]
    flat_idx = x.reshape(B * S)
    out_sdb = _make_gather(B, S, D)(table, flat_idx)
    return jnp.transpose(out_sdb, (2, 0, 1))


# speedup vs baseline: 1.1134x; 1.1134x over previous
import functools
import jax
import jax.numpy as jnp
from jax import lax
from jax.experimental import pallas as pl
from jax.experimental.pallas import tpu as pltpu, tpu_sc as plsc

VOCAB = 1000000
D_MODEL = 64
NC, NS = 2, 16
NW = NC * NS
L = 16


def _make_gather(B: int, S: int, D: int):
    assert B % NW == 0
    b_per_w = B // NW
    n_idx = b_per_w * S
    assert b_per_w % L == 0
    n_groups = b_per_w // L
    mesh = plsc.VectorSubcoreMesh(
        core_axis_name="c", subcore_axis_name="s",
        num_cores=NC, num_subcores=NS)

    assert D % 8 == 0 and b_per_w % 128 == 0
    n_dt = D // 8
    n_bt = b_per_w // 128

    @functools.partial(
        pl.kernel, mesh=mesh,
        out_type=jax.ShapeDtypeStruct((S, n_dt, B // 128, 8, 128),
                                      jnp.float32),
        compiler_params=pltpu.CompilerParams(
            use_tc_tiling_on_sc=False, needs_layout_passes=False),
        scratch_types=[
            pltpu.VMEM((n_idx,), jnp.int32),
            [pltpu.VMEM((b_per_w,), jnp.int32)] * 2,
            [pltpu.VMEM((b_per_w, D), jnp.float32)] * 2,
            pltpu.VMEM((n_dt, n_bt, 8, 128), jnp.float32),
            [pltpu.SemaphoreType.DMA] * 2,
            pltpu.SemaphoreType.DMA,
        ],
    )
    def k(table_hbm, idx_hbm, out_hbm, x_all, idx_s, rows, trans,
          gsem, ssem):
        wid = lax.axis_index("s") * NC + lax.axis_index("c")
        base = wid * b_per_w
        bt0 = wid * n_bt

        pltpu.sync_copy(idx_hbm.at[pl.ds(base * S, n_idx)], x_all)

        def build_idx(s, slot):
            offs0 = lax.iota(jnp.int32, L) * S
            for g in range(n_groups):
                offs = offs0 + (g * L * S + s)
                idx_s[slot][pl.ds(g * L, L)] = plsc.load_gather(
                    x_all, [offs])

        def gather(slot):
            return pltpu.async_copy(
                table_hbm.at[idx_s[slot]], rows[slot], gsem[slot])

        def gather_wait(slot):
            pltpu.make_async_copy(
                table_hbm.at[idx_s[slot]], rows[slot], gsem[slot]).wait()

        def transpose(slot):
            @pl.loop(0, n_dt * 8, unroll=4)
            def _(d):
                dt = d // 8
                dsub = d % 8
                b_iota = lax.iota(jnp.int32, L)
                d_splat = jnp.full((L,), d, jnp.int32)
                for bt in range(n_bt):
                    for g in range(128 // L):
                        trans[dt, bt, dsub, pl.ds(g * L, L)] = (
                            plsc.load_gather(
                                rows[slot],
                                [b_iota + (bt * 128 + g * L), d_splat]))

        def store(s):
            for dt in range(n_dt):
                pltpu.async_copy(
                    trans.at[dt],
                    out_hbm.at[s, dt, pl.ds(bt0, n_bt)], ssem)

        def store_wait(s):
            for dt in range(n_dt):
                pltpu.make_async_copy(
                    trans.at[dt],
                    out_hbm.at[s, dt, pl.ds(bt0, n_bt)], ssem).wait()

        build_idx(0, 0)
        gather(0)

        @pl.loop(0, S, step=2)
        def _(s0):
            for b in range(2):
                s = s0 + b
                q = 1 - b
                gather_wait(b)

                @pl.when(s + 1 < S)
                def _():
                    build_idx(s + 1, q)
                    gather(q)

                @pl.when(s > 0)
                def _():
                    store_wait(s - 1)
                transpose(b)
                store(s)

        store_wait(S - 1)

    return k


def kernel(x, table):
    B, S = x.shape
    D = table.shape[1]
    flat_idx = x.reshape(B * S)
    out_phys = _make_gather(B, S, D)(table, flat_idx)
    return jnp.transpose(out_phys, (2, 4, 0, 1, 3)).reshape(B, S, D)


# speedup vs baseline: 2.7786x; 2.4956x over previous
import functools
import jax
import jax.numpy as jnp
from jax import lax
from jax.experimental import pallas as pl
from jax.experimental.pallas import tpu as pltpu, tpu_sc as plsc

VOCAB = 1000000
D_MODEL = 64
NC, NS = 2, 16
NW = NC * NS
L = 16


def _make_gather(B: int, S: int, D: int):
    assert B % NW == 0
    b_per_w = B // NW
    n_idx = b_per_w * S
    assert b_per_w % L == 0
    n_groups = b_per_w // L
    mesh = plsc.VectorSubcoreMesh(
        core_axis_name="c", subcore_axis_name="s",
        num_cores=NC, num_subcores=NS)

    assert D % 8 == 0 and b_per_w % 128 == 0
    n_dt = D // 8
    n_bt = b_per_w // 128

    @functools.partial(
        pl.kernel, mesh=mesh,
        out_type=jax.ShapeDtypeStruct((S, n_dt, B // 128, 8, 128),
                                      jnp.float32),
        compiler_params=pltpu.CompilerParams(
            use_tc_tiling_on_sc=False, needs_layout_passes=False),
        scratch_types=[
            pltpu.VMEM((n_idx,), jnp.int32),
            [pltpu.VMEM((b_per_w,), jnp.int32)] * 2,
            [pltpu.VMEM((b_per_w, D), jnp.float32)] * 2,
            pltpu.VMEM((n_dt, n_bt, 8, 128), jnp.float32),
            [pltpu.SemaphoreType.DMA] * 2,
            pltpu.SemaphoreType.DMA,
        ],
    )
    def k(table_hbm, idx_hbm, out_hbm, x_all, idx_s, rows, trans,
          gsem, ssem):
        wid = lax.axis_index("s") * NC + lax.axis_index("c")
        base = wid * b_per_w
        bt0 = wid * n_bt

        pltpu.sync_copy(idx_hbm.at[pl.ds(base * S, n_idx)], x_all)

        def build_idx(s, slot):
            offs0 = lax.iota(jnp.int32, L) * S
            for g in range(n_groups):
                offs = offs0 + (g * L * S + s)
                idx_s[slot][pl.ds(g * L, L)] = plsc.load_gather(
                    x_all, [offs])

        def gather(slot):
            return pltpu.async_copy(
                table_hbm.at[idx_s[slot]], rows[slot], gsem[slot])

        def gather_wait(slot):
            pltpu.make_async_copy(
                table_hbm.at[idx_s[slot]], rows[slot], gsem[slot]).wait()

        def transpose(slot):
            @functools.partial(plsc.parallel_loop, 0, n_dt * 8, unroll=4)
            def _(d):
                dt = d // 8
                dsub = d % 8
                b_iota = lax.iota(jnp.int32, L)
                d_splat = jnp.full((L,), d, jnp.int32)
                for bt in range(n_bt):
                    for g in range(128 // L):
                        trans[dt, bt, dsub, pl.ds(g * L, L)] = (
                            plsc.load_gather(
                                rows[slot],
                                [b_iota + (bt * 128 + g * L), d_splat]))

        def store(s):
            for dt in range(n_dt):
                pltpu.async_copy(
                    trans.at[dt],
                    out_hbm.at[s, dt, pl.ds(bt0, n_bt)], ssem)

        def store_wait(s):
            for dt in range(n_dt):
                pltpu.make_async_copy(
                    trans.at[dt],
                    out_hbm.at[s, dt, pl.ds(bt0, n_bt)], ssem).wait()

        build_idx(0, 0)
        gather(0)

        @pl.loop(0, S, step=2)
        def _(s0):
            for b in range(2):
                s = s0 + b
                q = 1 - b
                gather_wait(b)

                @pl.when(s + 1 < S)
                def _():
                    build_idx(s + 1, q)
                    gather(q)

                @pl.when(s > 0)
                def _():
                    store_wait(s - 1)
                transpose(b)
                store(s)

        store_wait(S - 1)

    return k


def kernel(x, table):
    B, S = x.shape
    D = table.shape[1]
    flat_idx = x.reshape(B * S)
    out_phys = _make_gather(B, S, D)(table, flat_idx)
    return jnp.transpose(out_phys, (2, 4, 0, 1, 3)).reshape(B, S, D)
